# baseline (speedup 1.0000x reference)
import functools

import jax
import jax.numpy as jnp
from jax import lax
from jax.experimental import pallas as pl
from jax.experimental.pallas import tpu as pltpu
from jax.experimental.pallas import tpu_sc as plsc

_C = 1024
_E = 8
_D = 2048
_N = 4096
_TB = 256
_TM = 256
_CAP = _N * 2 + _E * _TM
_NT = _CAP // _TM
_NW = 32


def _fiota(shape, dim):
    return lax.broadcasted_iota(jnp.int32, shape, dim).astype(jnp.float32)


def _router_body(x_ref, wg_ref, oh1_ref, oh2_ref, r1_ref, r2_ref,
                 w1_ref, w2_ref, counts_ref):
    step = pl.program_id(0)

    @pl.when(step == 0)
    def _():
        counts_ref[...] = jnp.zeros_like(counts_ref)

    x = x_ref[...]
    wg = wg_ref[...]
    logits = jnp.dot(x, wg, preferred_element_type=jnp.float32)

    lane = _fiota((_TB, _E), 1)
    m1 = jnp.max(logits, axis=1, keepdims=True)
    i1 = jnp.min(jnp.where(logits >= m1, lane, float(_E)), axis=1,
                 keepdims=True)
    oh1 = (lane == i1).astype(jnp.float32)
    masked = jnp.where(oh1 > 0.0, -jnp.inf, logits)
    m2 = jnp.max(masked, axis=1, keepdims=True)
    i2 = jnp.min(jnp.where(masked >= m2, lane, float(_E)), axis=1,
                 keepdims=True)
    oh2 = (lane == i2).astype(jnp.float32)

    e2 = jnp.exp(m2 - m1)
    s = 1.0 + e2
    w1 = 1.0 / s
    w2 = e2 / s
    w1_ref[...] = jnp.broadcast_to(w1, (_TB, 16))
    w2_ref[...] = jnp.broadcast_to(w2, (_TB, 16))
    oh1_ref[...] = oh1
    oh2_ref[...] = oh2

    row_i = _fiota((_TB, _TB), 0)
    col_i = _fiota((_TB, _TB), 1)
    tril = (row_i > col_i).astype(jnp.float32)
    c0 = counts_ref[0:1, :]
    r1_ref[...] = jnp.dot(tril, oh1, preferred_element_type=jnp.float32) + c0
    c1 = c0 + jnp.sum(oh1, axis=0, keepdims=True)
    r2_ref[...] = jnp.dot(tril, oh2, preferred_element_type=jnp.float32) + c1
    counts_ref[0:1, :] = c1 + jnp.sum(oh2, axis=0, keepdims=True)


def _run_router(x2, wg):
    nblk = _N // _TB
    return pl.pallas_call(
        _router_body,
        grid=(nblk,),
        in_specs=[
            pl.BlockSpec((_TB, _C), lambda i: (i, 0)),
            pl.BlockSpec((_C, _E), lambda i: (0, 0)),
        ],
        out_specs=[
            pl.BlockSpec((_TB, _E), lambda i: (i, 0)),
            pl.BlockSpec((_TB, _E), lambda i: (i, 0)),
            pl.BlockSpec((_TB, _E), lambda i: (i, 0)),
            pl.BlockSpec((_TB, _E), lambda i: (i, 0)),
            pl.BlockSpec((_TB, 16), lambda i: (i, 0)),
            pl.BlockSpec((_TB, 16), lambda i: (i, 0)),
            pl.BlockSpec((8, _E), lambda i: (0, 0)),
        ],
        out_shape=[
            jax.ShapeDtypeStruct((_N, _E), jnp.float32),
            jax.ShapeDtypeStruct((_N, _E), jnp.float32),
            jax.ShapeDtypeStruct((_N, _E), jnp.float32),
            jax.ShapeDtypeStruct((_N, _E), jnp.float32),
            jax.ShapeDtypeStruct((_N, 16), jnp.float32),
            jax.ShapeDtypeStruct((_N, 16), jnp.float32),
            jax.ShapeDtypeStruct((8, _E), jnp.float32),
        ],
    )(x2, wg)


def _pos_body(oh1_ref, oh2_ref, r1_ref, r2_ref, counts_ref,
              p1_ref, p2_ref, te_ref):
    c = counts_ref[0:1, :]
    ct = jnp.floor((c + float(_TM - 1)) * (1.0 / _TM))
    ei = _fiota((_E, _E), 0)
    ej = _fiota((_E, _E), 1)
    mstrict = (ei < ej).astype(jnp.float32)
    cum_excl = jnp.dot(ct, mstrict, preferred_element_type=jnp.float32)
    poff = cum_excl * float(_TM)

    p1 = (r1_ref[...] + poff) * oh1_ref[...]
    p2 = (r2_ref[...] + poff) * oh2_ref[...]
    p1_ref[...] = jnp.broadcast_to(jnp.sum(p1, axis=1, keepdims=True),
                                   (_TB, _E))
    p2_ref[...] = jnp.broadcast_to(jnp.sum(p2, axis=1, keepdims=True),
                                   (_TB, _E))

    cum_incl = cum_excl + ct
    it = _fiota((48, _E), 0)
    cmp = (jnp.broadcast_to(cum_incl, (48, _E)) <= it).astype(jnp.float32)
    te_col = jnp.sum(cmp, axis=1, keepdims=True)
    te = jnp.minimum(te_col, float(_E - 1))
    te_ref[...] = jnp.broadcast_to(te, (48, _E))


def _run_pos(oh1, oh2, r1, r2, counts):
    nblk = _N // _TB
    return pl.pallas_call(
        _pos_body,
        grid=(nblk,),
        in_specs=[
            pl.BlockSpec((_TB, _E), lambda i: (i, 0)),
            pl.BlockSpec((_TB, _E), lambda i: (i, 0)),
            pl.BlockSpec((_TB, _E), lambda i: (i, 0)),
            pl.BlockSpec((_TB, _E), lambda i: (i, 0)),
            pl.BlockSpec((8, _E), lambda i: (0, 0)),
        ],
        out_specs=[
            pl.BlockSpec((_TB, _E), lambda i: (i, 0)),
            pl.BlockSpec((_TB, _E), lambda i: (i, 0)),
            pl.BlockSpec((48, _E), lambda i: (0, 0)),
        ],
        out_shape=[
            jax.ShapeDtypeStruct((_N, _E), jnp.float32),
            jax.ShapeDtypeStruct((_N, _E), jnp.float32),
            jax.ShapeDtypeStruct((48, _E), jnp.float32),
        ],
    )(oh1, oh2, r1, r2, counts)


_ROWS_PER_W = _CAP // _NW
_GCHUNK = 64


def _sc_dispatch_body(pos0_hbm, pos1_hbm, x_hbm, xs_hbm,
                      p0_v, p1_v, tok_v, row_a, row_b,
                      sem_ia, sem_ib, sem_oa, sem_ob):
    wid = lax.axis_index("s") * 2 + lax.axis_index("c")
    pltpu.sync_copy(pos0_hbm, p0_v)
    pltpu.sync_copy(pos1_hbm, p1_v)

    base = wid * _ROWS_PER_W
    zeros16 = jnp.zeros((16,), dtype=jnp.int32)

    def _zero(i, carry):
        tok_v[pl.ds(base + i * 16, 16)] = zeros16
        return carry

    lax.fori_loop(0, _ROWS_PER_W // 16, _zero, 0)

    iota16 = lax.iota(jnp.int32, 16)

    def _scat(cidx, carry):
        tvec = cidx * 16 + iota16
        plsc.store_scatter(tok_v, [p0_v[pl.ds(cidx * 16, 16)]], tvec)
        plsc.store_scatter(tok_v, [p1_v[pl.ds(cidx * 16, 16)]], tvec)
        return carry

    lax.fori_loop(0, _N // 16, _scat, 0)

    nchunks = _ROWS_PER_W // _GCHUNK
    bufs = [row_a, row_b]
    isems = [sem_ia, sem_ib]
    osems = [sem_oa, sem_ob]

    def _gather(k, buf, sem):
        idxs = tok_v.at[pl.ds(base + k * _GCHUNK, _GCHUNK)]
        return pltpu.async_copy(x_hbm.at[idxs], buf, sem)

    incopy = [None] * nchunks
    outcopy = [None] * nchunks
    incopy[0] = _gather(0, bufs[0], isems[0])
    for k in range(nchunks):
        cur = k % 2
        nxt = (k + 1) % 2
        incopy[k].wait()
        if k + 1 < nchunks:
            if k - 1 >= 0:
                outcopy[k - 1].wait()
            incopy[k + 1] = _gather(k + 1, bufs[nxt], isems[nxt])
        outcopy[k] = pltpu.async_copy(
            bufs[cur], xs_hbm.at[pl.ds(base + k * _GCHUNK, _GCHUNK)],
            osems[cur])
    outcopy[nchunks - 2].wait()
    outcopy[nchunks - 1].wait()


def _run_sc_dispatch(pos0, pos1, x3):
    mesh = plsc.VectorSubcoreMesh(core_axis_name="c", subcore_axis_name="s")
    f = pl.kernel(
        _sc_dispatch_body,
        out_type=jax.ShapeDtypeStruct((_CAP, _C // 2), jnp.int32),
        mesh=mesh,
        compiler_params=pltpu.CompilerParams(needs_layout_passes=False),
        scratch_types=[
            pltpu.VMEM((_N,), jnp.int32),
            pltpu.VMEM((_N,), jnp.int32),
            pltpu.VMEM((_CAP,), jnp.int32),
            pltpu.VMEM((_GCHUNK, _C // 2), jnp.int32),
            pltpu.VMEM((_GCHUNK, _C // 2), jnp.int32),
            pltpu.SemaphoreType.DMA,
            pltpu.SemaphoreType.DMA,
            pltpu.SemaphoreType.DMA,
            pltpu.SemaphoreType.DMA,
        ],
    )
    return f(pos0, pos1, x3)


def _gmm_body(te_ref, xs_ref, w1_ref, b1_ref, w2_ref, b2_ref, y_ref):
    x = xs_ref[...]
    h = jnp.dot(x, w1_ref[0], preferred_element_type=jnp.float32)
    h = h + b1_ref[0]
    h = jax.nn.gelu(h)
    y = jnp.dot(h.astype(jnp.bfloat16), w2_ref[0],
                preferred_element_type=jnp.float32)
    y_ref[...] = y + b2_ref[0]


def _run_gmm(te, xs, w1, b1, w2, b2):
    grid_spec = pltpu.PrefetchScalarGridSpec(
        num_scalar_prefetch=1,
        grid=(_NT,),
        in_specs=[
            pl.BlockSpec((_TM, _C), lambda i, te: (i, 0)),
            pl.BlockSpec((1, _C, _D), lambda i, te: (te[i], 0, 0)),
            pl.BlockSpec((1, 1, _D), lambda i, te: (te[i], 0, 0)),
            pl.BlockSpec((1, _D, _C), lambda i, te: (te[i], 0, 0)),
            pl.BlockSpec((1, 1, _C), lambda i, te: (te[i], 0, 0)),
        ],
        out_specs=pl.BlockSpec((_TM, _C), lambda i, te: (i, 0)),
    )
    return pl.pallas_call(
        _gmm_body,
        grid_spec=grid_spec,
        out_shape=jax.ShapeDtypeStruct((_CAP, _C), jnp.float32),
    )(te, xs, w1, b1, w2, b2)


_TPT = _N // _NW
_CCHUNK = 32


def _sc_combine_body(y_hbm, pos0_hbm, pos1_hbm, w1_hbm, w2_hbm, out_hbm,
                     p0_v, p1_v, w1_v, w2_v, buf0, buf1, sem0, sem1):
    wid = lax.axis_index("s") * 2 + lax.axis_index("c")
    tbase = wid * _TPT
    pltpu.sync_copy(pos0_hbm.at[pl.ds(tbase, _TPT)], p0_v)
    pltpu.sync_copy(pos1_hbm.at[pl.ds(tbase, _TPT)], p1_v)
    pltpu.sync_copy(w1_hbm.at[pl.ds(tbase, _TPT)], w1_v)
    pltpu.sync_copy(w2_hbm.at[pl.ds(tbase, _TPT)], w2_v)

    for k in range(_TPT // _CCHUNK):
        rbase = k * _CCHUNK
        cp0 = pltpu.async_copy(
            y_hbm.at[p0_v.at[pl.ds(rbase, _CCHUNK)]], buf0, sem0)
        cp1 = pltpu.async_copy(
            y_hbm.at[p1_v.at[pl.ds(rbase, _CCHUNK)]], buf1, sem1)
        cp0.wait()
        cp1.wait()

        def _row(r, carry):
            g0 = w1_v[rbase + r]
            g1 = w2_v[rbase + r]
            for cc in range(_C // 16):
                a = buf0[r, pl.ds(cc * 16, 16)]
                b = buf1[r, pl.ds(cc * 16, 16)]
                buf0[r, pl.ds(cc * 16, 16)] = g0 * a + g1 * b
            return carry

        lax.fori_loop(0, _CCHUNK, _row, 0)
        pltpu.sync_copy(buf0, out_hbm.at[pl.ds(tbase + rbase, _CCHUNK)])


def _run_sc_combine(y, pos0, pos1, w1b, w2b):
    mesh = plsc.VectorSubcoreMesh(core_axis_name="c", subcore_axis_name="s")
    f = pl.kernel(
        _sc_combine_body,
        out_type=jax.ShapeDtypeStruct((_N, _C), jnp.float32),
        mesh=mesh,
        compiler_params=pltpu.CompilerParams(needs_layout_passes=False),
        scratch_types=[
            pltpu.VMEM((_TPT,), jnp.int32),
            pltpu.VMEM((_TPT,), jnp.int32),
            pltpu.VMEM((_TPT, 16), jnp.float32),
            pltpu.VMEM((_TPT, 16), jnp.float32),
            pltpu.VMEM((_CCHUNK, _C), jnp.float32),
            pltpu.VMEM((_CCHUNK, _C), jnp.float32),
            pltpu.SemaphoreType.DMA,
            pltpu.SemaphoreType.DMA,
        ],
    )
    return f(y, pos0, pos1, w1b, w2b)


def kernel(X, Wg, W1, b1, W2, b2):
    Bx, Tx, C = X.shape
    x2 = X.reshape(-1, C)

    oh1, oh2, r1, r2, w1b, w2b, counts = _run_router(x2, Wg)
    p1b, p2b, teb = _run_pos(oh1, oh2, r1, r2, counts)

    pos0 = p1b[:, 0].astype(jnp.int32)
    pos1 = p2b[:, 0].astype(jnp.int32)
    te = teb[:_NT, 0].astype(jnp.int32)

    x3 = lax.bitcast_convert_type(
        x2.astype(jnp.bfloat16).reshape(_N, _C // 2, 2), jnp.int32)
    xs_i = _run_sc_dispatch(pos0, pos1, x3)
    xs = lax.bitcast_convert_type(xs_i, jnp.bfloat16).reshape(_CAP, _C)
    y = _run_gmm(te, xs,
                 W1.astype(jnp.bfloat16),
                 b1.reshape(_E, 1, _D),
                 W2.astype(jnp.bfloat16),
                 b2.reshape(_E, 1, _C))
    out = _run_sc_combine(y, pos0, pos1, w1b, w2b)
    return out.reshape(Bx, Tx, C)


# speedup vs baseline: 1.7543x; 1.7543x over previous
import functools

import jax
import jax.numpy as jnp
from jax import lax
from jax.experimental import pallas as pl
from jax.experimental.pallas import tpu as pltpu
from jax.experimental.pallas import tpu_sc as plsc

_C = 1024
_E = 8
_D = 2048
_N = 4096
_TB = 256
_TM = 256
_CAP = _N * 2 + _E * _TM
_NT = _CAP // _TM
_NW = 32


def _fiota(shape, dim):
    return lax.broadcasted_iota(jnp.int32, shape, dim).astype(jnp.float32)


def _router_body(x_ref, wg_ref, oh1_ref, oh2_ref, r1_ref, r2_ref,
                 w1_ref, w2_ref, counts_ref, xp_ref):
    step = pl.program_id(0)

    @pl.when(step == 0)
    def _():
        counts_ref[...] = jnp.zeros_like(counts_ref)

    x = x_ref[...]
    wg = wg_ref[...]
    logits = jnp.dot(x, wg, preferred_element_type=jnp.float32)

    xb = x.astype(jnp.bfloat16)
    lo = lax.bitcast_convert_type(xb[:, :_C // 2], jnp.uint16)
    hi = lax.bitcast_convert_type(xb[:, _C // 2:], jnp.uint16)
    word = lo.astype(jnp.uint32) | (hi.astype(jnp.uint32) << 16)
    xp_ref[...] = lax.bitcast_convert_type(word, jnp.int32)

    lane = _fiota((_TB, _E), 1)
    m1 = jnp.max(logits, axis=1, keepdims=True)
    i1 = jnp.min(jnp.where(logits >= m1, lane, float(_E)), axis=1,
                 keepdims=True)
    oh1 = (lane == i1).astype(jnp.float32)
    masked = jnp.where(oh1 > 0.0, -jnp.inf, logits)
    m2 = jnp.max(masked, axis=1, keepdims=True)
    i2 = jnp.min(jnp.where(masked >= m2, lane, float(_E)), axis=1,
                 keepdims=True)
    oh2 = (lane == i2).astype(jnp.float32)

    e2 = jnp.exp(m2 - m1)
    s = 1.0 + e2
    w1 = 1.0 / s
    w2 = e2 / s
    w1_ref[...] = jnp.broadcast_to(w1, (_TB, 16))
    w2_ref[...] = jnp.broadcast_to(w2, (_TB, 16))
    oh1_ref[...] = oh1
    oh2_ref[...] = oh2

    row_i = _fiota((_TB, _TB), 0)
    col_i = _fiota((_TB, _TB), 1)
    tril = (row_i > col_i).astype(jnp.float32)
    c0 = counts_ref[0:1, :]
    r1_ref[...] = jnp.dot(tril, oh1, preferred_element_type=jnp.float32) + c0
    c1 = c0 + jnp.sum(oh1, axis=0, keepdims=True)
    r2_ref[...] = jnp.dot(tril, oh2, preferred_element_type=jnp.float32) + c1
    counts_ref[0:1, :] = c1 + jnp.sum(oh2, axis=0, keepdims=True)


def _run_router(x2, wg):
    nblk = _N // _TB
    return pl.pallas_call(
        _router_body,
        grid=(nblk,),
        in_specs=[
            pl.BlockSpec((_TB, _C), lambda i: (i, 0)),
            pl.BlockSpec((_C, _E), lambda i: (0, 0)),
        ],
        out_specs=[
            pl.BlockSpec((_TB, _E), lambda i: (i, 0)),
            pl.BlockSpec((_TB, _E), lambda i: (i, 0)),
            pl.BlockSpec((_TB, _E), lambda i: (i, 0)),
            pl.BlockSpec((_TB, _E), lambda i: (i, 0)),
            pl.BlockSpec((_TB, 16), lambda i: (i, 0)),
            pl.BlockSpec((_TB, 16), lambda i: (i, 0)),
            pl.BlockSpec((8, _E), lambda i: (0, 0)),
            pl.BlockSpec((_TB, _C // 2), lambda i: (i, 0)),
        ],
        out_shape=[
            jax.ShapeDtypeStruct((_N, _E), jnp.float32),
            jax.ShapeDtypeStruct((_N, _E), jnp.float32),
            jax.ShapeDtypeStruct((_N, _E), jnp.float32),
            jax.ShapeDtypeStruct((_N, _E), jnp.float32),
            jax.ShapeDtypeStruct((_N, 16), jnp.float32),
            jax.ShapeDtypeStruct((_N, 16), jnp.float32),
            jax.ShapeDtypeStruct((8, _E), jnp.float32),
            jax.ShapeDtypeStruct((_N, _C // 2), jnp.int32),
        ],
    )(x2, wg)


def _pos_body(oh1_ref, oh2_ref, r1_ref, r2_ref, counts_ref,
              p1_ref, p2_ref, te_ref):
    c = counts_ref[0:1, :]
    ct = jnp.floor((c + float(_TM - 1)) * (1.0 / _TM))
    ei = _fiota((_E, _E), 0)
    ej = _fiota((_E, _E), 1)
    mstrict = (ei < ej).astype(jnp.float32)
    cum_excl = jnp.dot(ct, mstrict, preferred_element_type=jnp.float32)
    poff = cum_excl * float(_TM)

    p1 = (r1_ref[...] + poff) * oh1_ref[...]
    p2 = (r2_ref[...] + poff) * oh2_ref[...]
    p1_ref[...] = jnp.broadcast_to(jnp.sum(p1, axis=1, keepdims=True),
                                   (_TB, _E))
    p2_ref[...] = jnp.broadcast_to(jnp.sum(p2, axis=1, keepdims=True),
                                   (_TB, _E))

    cum_incl = cum_excl + ct
    it = _fiota((48, _E), 0)
    cmp = (jnp.broadcast_to(cum_incl, (48, _E)) <= it).astype(jnp.float32)
    te_col = jnp.sum(cmp, axis=1, keepdims=True)
    te = jnp.minimum(te_col, float(_E - 1))
    te_ref[...] = jnp.broadcast_to(te, (48, _E))


def _run_pos(oh1, oh2, r1, r2, counts):
    nblk = _N // _TB
    return pl.pallas_call(
        _pos_body,
        grid=(nblk,),
        in_specs=[
            pl.BlockSpec((_TB, _E), lambda i: (i, 0)),
            pl.BlockSpec((_TB, _E), lambda i: (i, 0)),
            pl.BlockSpec((_TB, _E), lambda i: (i, 0)),
            pl.BlockSpec((_TB, _E), lambda i: (i, 0)),
            pl.BlockSpec((8, _E), lambda i: (0, 0)),
        ],
        out_specs=[
            pl.BlockSpec((_TB, _E), lambda i: (i, 0)),
            pl.BlockSpec((_TB, _E), lambda i: (i, 0)),
            pl.BlockSpec((48, _E), lambda i: (0, 0)),
        ],
        out_shape=[
            jax.ShapeDtypeStruct((_N, _E), jnp.float32),
            jax.ShapeDtypeStruct((_N, _E), jnp.float32),
            jax.ShapeDtypeStruct((48, _E), jnp.float32),
        ],
    )(oh1, oh2, r1, r2, counts)


_ROWS_PER_W = _CAP // _NW
_GCHUNK = 64


def _sc_dispatch_body(pos0_hbm, pos1_hbm, x_hbm, xs_hbm,
                      p0_v, p1_v, tok_v, row_a, row_b,
                      sem_ia, sem_ib, sem_oa, sem_ob):
    wid = lax.axis_index("s") * 2 + lax.axis_index("c")
    pltpu.sync_copy(pos0_hbm, p0_v)
    pltpu.sync_copy(pos1_hbm, p1_v)

    base = wid * _ROWS_PER_W
    zeros16 = jnp.zeros((16,), dtype=jnp.int32)

    def _zero(i, carry):
        tok_v[pl.ds(base + i * 16, 16)] = zeros16
        return carry

    lax.fori_loop(0, _ROWS_PER_W // 16, _zero, 0)

    iota16 = lax.iota(jnp.int32, 16)

    def _scat(cidx, carry):
        tvec = cidx * 16 + iota16
        plsc.store_scatter(tok_v, [p0_v[pl.ds(cidx * 16, 16)]], tvec)
        plsc.store_scatter(tok_v, [p1_v[pl.ds(cidx * 16, 16)]], tvec)
        return carry

    lax.fori_loop(0, _N // 16, _scat, 0)

    nchunks = _ROWS_PER_W // _GCHUNK
    bufs = [row_a, row_b]
    isems = [sem_ia, sem_ib]
    osems = [sem_oa, sem_ob]

    def _gather(k, buf, sem):
        idxs = tok_v.at[pl.ds(base + k * _GCHUNK, _GCHUNK)]
        return pltpu.async_copy(x_hbm.at[idxs], buf, sem)

    incopy = [None] * nchunks
    outcopy = [None] * nchunks
    incopy[0] = _gather(0, bufs[0], isems[0])
    for k in range(nchunks):
        cur = k % 2
        nxt = (k + 1) % 2
        incopy[k].wait()
        if k + 1 < nchunks:
            if k - 1 >= 0:
                outcopy[k - 1].wait()
            incopy[k + 1] = _gather(k + 1, bufs[nxt], isems[nxt])
        outcopy[k] = pltpu.async_copy(
            bufs[cur], xs_hbm.at[pl.ds(base + k * _GCHUNK, _GCHUNK)],
            osems[cur])
    outcopy[nchunks - 2].wait()
    outcopy[nchunks - 1].wait()


def _run_sc_dispatch(pos0, pos1, x3):
    mesh = plsc.VectorSubcoreMesh(core_axis_name="c", subcore_axis_name="s")
    f = pl.kernel(
        _sc_dispatch_body,
        out_type=jax.ShapeDtypeStruct((_CAP, _C // 2), jnp.int32),
        mesh=mesh,
        compiler_params=pltpu.CompilerParams(needs_layout_passes=False),
        scratch_types=[
            pltpu.VMEM((_N,), jnp.int32),
            pltpu.VMEM((_N,), jnp.int32),
            pltpu.VMEM((_CAP,), jnp.int32),
            pltpu.VMEM((_GCHUNK, _C // 2), jnp.int32),
            pltpu.VMEM((_GCHUNK, _C // 2), jnp.int32),
            pltpu.SemaphoreType.DMA,
            pltpu.SemaphoreType.DMA,
            pltpu.SemaphoreType.DMA,
            pltpu.SemaphoreType.DMA,
        ],
    )
    return f(pos0, pos1, x3)


def _gmm_body(te_ref, xs_ref, w1_ref, b1_ref, w2_ref, b2_ref, y_ref):
    xu = lax.bitcast_convert_type(xs_ref[...], jnp.uint32)
    lo = lax.bitcast_convert_type((xu & 0xffff).astype(jnp.uint16),
                                  jnp.bfloat16)
    hi = lax.bitcast_convert_type((xu >> 16).astype(jnp.uint16),
                                  jnp.bfloat16)
    x = jnp.concatenate([lo, hi], axis=1)
    h = jnp.dot(x, w1_ref[0], preferred_element_type=jnp.float32)
    h = h + b1_ref[0]
    h = jax.nn.gelu(h)
    y = jnp.dot(h.astype(jnp.bfloat16), w2_ref[0],
                preferred_element_type=jnp.float32)
    y_ref[...] = y + b2_ref[0]


def _run_gmm(te, xs, w1, b1, w2, b2):
    grid_spec = pltpu.PrefetchScalarGridSpec(
        num_scalar_prefetch=1,
        grid=(_NT,),
        in_specs=[
            pl.BlockSpec((_TM, _C // 2), lambda i, te: (i, 0)),
            pl.BlockSpec((1, _C, _D), lambda i, te: (te[i], 0, 0)),
            pl.BlockSpec((1, 1, _D), lambda i, te: (te[i], 0, 0)),
            pl.BlockSpec((1, _D, _C), lambda i, te: (te[i], 0, 0)),
            pl.BlockSpec((1, 1, _C), lambda i, te: (te[i], 0, 0)),
        ],
        out_specs=pl.BlockSpec((_TM, _C), lambda i, te: (i, 0)),
    )
    return pl.pallas_call(
        _gmm_body,
        grid_spec=grid_spec,
        out_shape=jax.ShapeDtypeStruct((_CAP, _C), jnp.float32),
    )(te, xs, w1, b1, w2, b2)


_TPT = _N // _NW
_CCHUNK = 32


def _sc_combine_body(y_hbm, pos0_hbm, pos1_hbm, w1_hbm, w2_hbm, out_hbm,
                     p0_v, p1_v, w1_v, w2_v, buf0, buf1, sem0, sem1):
    wid = lax.axis_index("s") * 2 + lax.axis_index("c")
    tbase = wid * _TPT
    pltpu.sync_copy(pos0_hbm.at[pl.ds(tbase, _TPT)], p0_v)
    pltpu.sync_copy(pos1_hbm.at[pl.ds(tbase, _TPT)], p1_v)
    pltpu.sync_copy(w1_hbm.at[pl.ds(tbase, _TPT)], w1_v)
    pltpu.sync_copy(w2_hbm.at[pl.ds(tbase, _TPT)], w2_v)

    for k in range(_TPT // _CCHUNK):
        rbase = k * _CCHUNK
        cp0 = pltpu.async_copy(
            y_hbm.at[p0_v.at[pl.ds(rbase, _CCHUNK)]], buf0, sem0)
        cp1 = pltpu.async_copy(
            y_hbm.at[p1_v.at[pl.ds(rbase, _CCHUNK)]], buf1, sem1)
        cp0.wait()
        cp1.wait()

        def _row(r, carry):
            g0 = w1_v[rbase + r]
            g1 = w2_v[rbase + r]
            for cc in range(_C // 16):
                a = buf0[r, pl.ds(cc * 16, 16)]
                b = buf1[r, pl.ds(cc * 16, 16)]
                buf0[r, pl.ds(cc * 16, 16)] = g0 * a + g1 * b
            return carry

        lax.fori_loop(0, _CCHUNK, _row, 0)
        pltpu.sync_copy(buf0, out_hbm.at[pl.ds(tbase + rbase, _CCHUNK)])


def _run_sc_combine(y, pos0, pos1, w1b, w2b):
    mesh = plsc.VectorSubcoreMesh(core_axis_name="c", subcore_axis_name="s")
    f = pl.kernel(
        _sc_combine_body,
        out_type=jax.ShapeDtypeStruct((_N, _C), jnp.float32),
        mesh=mesh,
        compiler_params=pltpu.CompilerParams(needs_layout_passes=False),
        scratch_types=[
            pltpu.VMEM((_TPT,), jnp.int32),
            pltpu.VMEM((_TPT,), jnp.int32),
            pltpu.VMEM((_TPT, 16), jnp.float32),
            pltpu.VMEM((_TPT, 16), jnp.float32),
            pltpu.VMEM((_CCHUNK, _C), jnp.float32),
            pltpu.VMEM((_CCHUNK, _C), jnp.float32),
            pltpu.SemaphoreType.DMA,
            pltpu.SemaphoreType.DMA,
        ],
    )
    return f(y, pos0, pos1, w1b, w2b)


def kernel(X, Wg, W1, b1, W2, b2):
    Bx, Tx, C = X.shape
    x2 = X.reshape(-1, C)

    oh1, oh2, r1, r2, w1b, w2b, counts, xp = _run_router(x2, Wg)
    p1b, p2b, teb = _run_pos(oh1, oh2, r1, r2, counts)

    pos0 = p1b[:, 0].astype(jnp.int32)
    pos1 = p2b[:, 0].astype(jnp.int32)
    te = teb[:_NT, 0].astype(jnp.int32)

    xs = _run_sc_dispatch(pos0, pos1, xp)
    y = _run_gmm(te, xs,
                 W1.astype(jnp.bfloat16),
                 b1.reshape(_E, 1, _D),
                 W2.astype(jnp.bfloat16),
                 b2.reshape(_E, 1, _C))
    out = _run_sc_combine(y, pos0, pos1, w1b, w2b)
    return out.reshape(Bx, Tx, C)
